# baseline (speedup 1.0000x reference)
import functools

import jax
import jax.numpy as jnp
from jax import lax
from jax.experimental import pallas as pl
from jax.experimental.pallas import tpu as pltpu, tpu_sc as plsc

H = 192
NA = 5
YW = NA * H
YWP = 1024
NC, NS, L = 2, 16, 16
NW = NC * NS
NCHUNK = H // L
HP = 256


def _prep_body(u_ref, w1_ref, b1p_ref, rmask_ref, w2_ref, b2_ref, out_ref):
    a5 = jnp.dot(u_ref[...], w1_ref[...],
                 preferred_element_type=jnp.float32) + b1p_ref[...]
    m = jnp.dot(a5, w2_ref[...], preferred_element_type=jnp.float32)
    out_ref[...] = m + rmask_ref[...] * b2_ref[...]


def _prep_layer(upad, w1, b1pad, rowmask, w2, b2row):
    nb = 8
    bc = (H * H) // nb
    return pl.pallas_call(
        _prep_body,
        grid=(nb,),
        in_specs=[
            pl.BlockSpec((8, H), lambda j: (0, 0)),
            pl.BlockSpec((H, H), lambda j: (0, 0)),
            pl.BlockSpec((8, H), lambda j: (0, 0)),
            pl.BlockSpec((8, 1), lambda j: (0, 0)),
            pl.BlockSpec((H, bc), lambda j: (0, j)),
            pl.BlockSpec((1, bc), lambda j: (0, j)),
        ],
        out_specs=pl.BlockSpec((8, bc), lambda j: (0, j)),
        out_shape=jax.ShapeDtypeStruct((8, H * H), jnp.float32),
    )(upad, w1, b1pad, rowmask, w2, b2row)


def _edge_sc(ycat, srcv, dstv, eaexp, zeros_nh, E):
    b_per_w = E // NW
    mesh = plsc.VectorSubcoreMesh(core_axis_name="c", subcore_axis_name="s")
    rows_per_sid = ycat.shape[0] // NS

    @functools.partial(
        pl.kernel, mesh=mesh,
        compiler_params=pltpu.CompilerParams(use_tc_tiling_on_sc=False),
        out_type=jax.ShapeDtypeStruct((NC, zeros_nh.shape[0], HP), jnp.float32),
        scratch_types=[
            pltpu.VMEM((b_per_w,), jnp.int32),
            pltpu.VMEM((b_per_w,), jnp.int32),
            pltpu.VMEM((b_per_w, YW), jnp.float32),
            pltpu.VMEM((b_per_w, YWP), jnp.float32),
            pltpu.VMEM((b_per_w, HP), jnp.float32),
            pltpu.VMEM_SHARED((zeros_nh.shape[0], HP), jnp.float32),
            pltpu.SemaphoreType.DMA,
        ],
    )
    def k(ycat_hbm, src_hbm, dst_hbm, ea_hbm, zero_hbm, out_hbm,
          src_v, dst_v, eax_v, rows_v, msg_v, acc_sh, sem):
        cid = lax.axis_index("c")
        sid = lax.axis_index("s")
        wid = sid * NC + cid
        base = wid * b_per_w

        pltpu.sync_copy(src_hbm.at[pl.ds(base, b_per_w)], src_v)
        pltpu.sync_copy(dst_hbm.at[pl.ds(base, b_per_w)], dst_v)
        pltpu.sync_copy(ea_hbm.at[pl.ds(base, b_per_w)], eax_v)
        gather = pltpu.async_copy(ycat_hbm.at[src_v], rows_v, sem)

        rbase = sid * rows_per_sid
        pltpu.sync_copy(zero_hbm.at[pl.ds(rbase, rows_per_sid)],
                        acc_sh.at[pl.ds(rbase, rows_per_sid)])
        gather.wait()

        def body(e, _):
            for c in range(NCHUNK):
                acc = (eax_v[e, pl.ds(c * L, L)] *
                       rows_v[e, pl.ds(c * L, L)])
                for a in range(1, NA):
                    o = a * H + c * L
                    acc = acc + (eax_v[e, pl.ds(o, L)] *
                                 rows_v[e, pl.ds(o, L)])
                msg_v[e, pl.ds(c * L, L)] = acc
            zero = jnp.zeros((L,), jnp.float32)
            for c in range(NCHUNK, HP // L):
                msg_v[e, pl.ds(c * L, L)] = zero
            return 0

        lax.fori_loop(0, b_per_w, body, 0)

        plsc.subcore_barrier()
        pltpu.sync_copy(msg_v, acc_sh.at[dst_v], add=True)
        plsc.subcore_barrier()
        pltpu.sync_copy(acc_sh.at[pl.ds(rbase, rows_per_sid)],
                        out_hbm.at[cid, pl.ds(rbase, rows_per_sid)])

    return k(ycat, srcv, dstv, eaexp, zeros_nh)


def _cheb_body(x_ref, src_ref, dst_ref, ea5_ref, chebw_ref, chebb_ref,
               m3_ref, root_ref, bias_ref, ycat_ref, base_ref, eaexp_ref):
    E = src_ref.shape[0]
    N = x_ref.shape[0]
    f32 = jnp.float32
    col = lax.broadcasted_iota(jnp.int32, (E, N), 1)
    G = (src_ref[...] == col).astype(f32)
    S = (dst_ref[...] == col).astype(f32)

    deg = jnp.sum(G, axis=0).reshape(N, 1)
    dis = jnp.where(deg > 0, lax.rsqrt(jnp.maximum(deg, 1e-12)), 0.0)
    norm = -(jnp.dot(G, dis, preferred_element_type=f32) *
             jnp.dot(S, dis, preferred_element_type=f32))

    def lhat(y):
        t = norm * jnp.dot(G, y, preferred_element_type=f32)
        return lax.dot_general(S, t, (((0,), (0,)), ((), ())),
                               preferred_element_type=f32)

    tx0 = x_ref[...]
    tx1 = lhat(tx0)
    tx2 = 2.0 * lhat(tx1) - tx0
    tx3 = 2.0 * lhat(tx2) - tx1
    tx4 = 2.0 * lhat(tx3) - tx2
    txcat = jnp.concatenate([tx0, tx1, tx2, tx3, tx4], axis=1)
    xc = jnp.dot(txcat, chebw_ref[...],
                 preferred_element_type=f32) + chebb_ref[...]

    ycat_ref[...] = jnp.dot(xc, m3_ref[...], preferred_element_type=f32)
    base_ref[...] = jnp.dot(xc, root_ref[...],
                            preferred_element_type=f32) + bias_ref[...]
    ea5 = ea5_ref[...]
    eaexp_ref[...] = jnp.concatenate(
        [jnp.broadcast_to(ea5[:, a:a + 1], (E, H)) for a in range(NA)],
        axis=1)


def _mid_body(base_ref, p_ref, gamma_ref, beta_ref, m3_ref, root_ref,
              bias_ref, ycat_ref, baseo_ref):
    f32 = jnp.float32
    xl = base_ref[...] + p_ref[0, :, :H] + p_ref[1, :, :H]
    mu = jnp.mean(xl, axis=1, keepdims=True)
    var = jnp.mean((xl - mu) ** 2, axis=1, keepdims=True)
    h = jnp.maximum((xl - mu) * lax.rsqrt(var + 1e-5) * gamma_ref[...]
                    + beta_ref[...], 0.0)
    ycat_ref[...] = jnp.dot(h, m3_ref[...], preferred_element_type=f32)
    baseo_ref[...] = xl + jnp.dot(h, root_ref[...],
                                  preferred_element_type=f32) + bias_ref[...]


def _final_body(base_ref, p_ref, gamma_ref, beta_ref, linw_ref, linb_ref,
                out_ref):
    f32 = jnp.float32
    xl = base_ref[...] + p_ref[0, :, :H] + p_ref[1, :, :H]
    mu = jnp.mean(xl, axis=1, keepdims=True)
    var = jnp.mean((xl - mu) ** 2, axis=1, keepdims=True)
    h = jnp.maximum((xl - mu) * lax.rsqrt(var + 1e-5) * gamma_ref[...]
                    + beta_ref[...], 0.0)
    out_ref[...] = jnp.dot(h, linw_ref[...],
                           preferred_element_type=f32) + linb_ref[...]


def kernel(x, edge_index, edge_attr, batch, params):
    N = x.shape[0]
    E = edge_index.shape[1]
    layers = params['layers']
    f32 = jnp.float32

    upad = jnp.zeros((8, H), f32)
    upad = upad.at[0:4, :].set(params['enc_W'])
    upad = upad.at[4, :].set(params['enc_b'])
    rowmask = jnp.zeros((8, 1), f32).at[4, 0].set(1.0)
    m3 = []
    for l in layers:
        b1pad = jnp.zeros((8, H), f32).at[4, :].set(l['nn_b1'])
        m = _prep_layer(upad, l['nn_W1'], b1pad, rowmask,
                        l['nn_W2'], l['nn_b2'].reshape(1, H * H))
        mt = jnp.transpose(m[:NA].reshape(NA, H, H), (1, 0, 2)).reshape(H, YW)
        m3.append(jnp.concatenate([mt, jnp.zeros((H, YWP - YW), f32)], axis=1))

    src = edge_index[0].astype(jnp.int32)
    dst = edge_index[1].astype(jnp.int32)
    ea5 = jnp.concatenate([edge_attr, jnp.ones((E, 1), f32)], axis=1)
    zeros_nh = jnp.zeros((N, HP), f32)
    chebw = params['cheb_W'].reshape(NA * 4, H)
    chebb = params['cheb_b'].reshape(1, H)

    l0 = layers[0]
    ycat, base, eaexp = pl.pallas_call(
        _cheb_body,
        out_shape=[jax.ShapeDtypeStruct((N, YWP), f32),
                   jax.ShapeDtypeStruct((N, H), f32),
                   jax.ShapeDtypeStruct((E, YW), f32)],
    )(x, src.reshape(E, 1), dst.reshape(E, 1), ea5, chebw, chebb, m3[0],
      l0['root'], l0['bias'].reshape(1, H))

    for l in (1, 2, 3):
        part = _edge_sc(ycat, src, dst, eaexp, zeros_nh, E)
        ll = layers[l]
        ycat, base = pl.pallas_call(
            _mid_body,
            out_shape=[jax.ShapeDtypeStruct((N, YWP), f32),
                       jax.ShapeDtypeStruct((N, H), f32)],
        )(base, part, ll['gamma'].reshape(1, H), ll['beta'].reshape(1, H),
          m3[l], ll['root'], ll['bias'].reshape(1, H))

    part = _edge_sc(ycat, src, dst, eaexp, zeros_nh, E)
    return pl.pallas_call(
        _final_body,
        out_shape=jax.ShapeDtypeStruct((N, 2), f32),
    )(base, part, l0['gamma'].reshape(1, H), l0['beta'].reshape(1, H),
      params['lin_W'], params['lin_b'].reshape(1, 2))


# speedup vs baseline: 1.0711x; 1.0711x over previous
import functools

import jax
import jax.numpy as jnp
from jax import lax
from jax.experimental import pallas as pl
from jax.experimental.pallas import tpu as pltpu, tpu_sc as plsc

H = 192
NA = 5
YW = NA * H
YWP = 1024
NC, NS, L = 2, 16, 16
NW = NC * NS
NCHUNK = H // L


def _prep_body(u_ref, w1_ref, b1p_ref, rmask_ref, w2_ref, b2_ref, out_ref):
    a5 = jnp.dot(u_ref[...], w1_ref[...],
                 preferred_element_type=jnp.float32) + b1p_ref[...]
    m = jnp.dot(a5, w2_ref[...], preferred_element_type=jnp.float32)
    out_ref[...] = m + rmask_ref[...] * b2_ref[...]


def _prep_layer(upad, w1, b1pad, rowmask, w2, b2row):
    nb = 8
    bc = (H * H) // nb
    return pl.pallas_call(
        _prep_body,
        grid=(nb,),
        in_specs=[
            pl.BlockSpec((8, H), lambda j: (0, 0)),
            pl.BlockSpec((H, H), lambda j: (0, 0)),
            pl.BlockSpec((8, H), lambda j: (0, 0)),
            pl.BlockSpec((8, 1), lambda j: (0, 0)),
            pl.BlockSpec((H, bc), lambda j: (0, j)),
            pl.BlockSpec((1, bc), lambda j: (0, j)),
        ],
        out_specs=pl.BlockSpec((8, bc), lambda j: (0, j)),
        out_shape=jax.ShapeDtypeStruct((8, H * H), jnp.float32),
    )(upad, w1, b1pad, rowmask, w2, b2row)


def _edge_sc(ycat, srcv, easpl, E):
    b_per_w = E // NW
    mesh = plsc.VectorSubcoreMesh(core_axis_name="c", subcore_axis_name="s")

    @functools.partial(
        pl.kernel, mesh=mesh,
        compiler_params=pltpu.CompilerParams(use_tc_tiling_on_sc=False),
        out_type=jax.ShapeDtypeStruct((E, H), jnp.float32),
        scratch_types=[
            pltpu.VMEM((b_per_w,), jnp.int32),
            pltpu.VMEM((b_per_w, NA * L), jnp.float32),
            pltpu.VMEM((b_per_w, YWP), jnp.float32),
            pltpu.VMEM((b_per_w, H), jnp.float32),
            pltpu.SemaphoreType.DMA,
        ],
    )
    def k(ycat_hbm, src_hbm, ea_hbm, out_hbm,
          src_v, eas_v, rows_v, msg_v, sem):
        cid = lax.axis_index("c")
        sid = lax.axis_index("s")
        wid = sid * NC + cid
        base = wid * b_per_w

        pltpu.sync_copy(src_hbm.at[pl.ds(base, b_per_w)], src_v)
        gather = pltpu.async_copy(ycat_hbm.at[src_v], rows_v, sem)
        pltpu.sync_copy(ea_hbm.at[pl.ds(base, b_per_w)], eas_v)
        gather.wait()

        def body(e, _):
            c0 = eas_v[e, pl.ds(0, L)]
            c1 = eas_v[e, pl.ds(L, L)]
            c2 = eas_v[e, pl.ds(2 * L, L)]
            c3 = eas_v[e, pl.ds(3 * L, L)]
            c4 = eas_v[e, pl.ds(4 * L, L)]
            for c in range(NCHUNK):
                o = c * L
                acc = (c0 * rows_v[e, pl.ds(o, L)] +
                       c1 * rows_v[e, pl.ds(H + o, L)] +
                       c2 * rows_v[e, pl.ds(2 * H + o, L)] +
                       c3 * rows_v[e, pl.ds(3 * H + o, L)] +
                       c4 * rows_v[e, pl.ds(4 * H + o, L)])
                msg_v[e, pl.ds(o, L)] = acc
            return 0

        lax.fori_loop(0, b_per_w, body, 0)
        pltpu.sync_copy(msg_v, out_hbm.at[pl.ds(base, b_per_w)])

    return k(ycat, srcv, easpl)


def _cheb_body(x_ref, src_ref, dst_ref, ea5_ref, chebw_ref, chebb_ref,
               m3_ref, root_ref, bias_ref, ycat_ref, base_ref, easpl_ref):
    E = src_ref.shape[0]
    N = x_ref.shape[0]
    f32 = jnp.float32
    col = lax.broadcasted_iota(jnp.int32, (E, N), 1)
    G = (src_ref[...] == col).astype(f32)
    S = (dst_ref[...] == col).astype(f32)

    deg = jnp.sum(G, axis=0).reshape(N, 1)
    dis = jnp.where(deg > 0, lax.rsqrt(jnp.maximum(deg, 1e-12)), 0.0)
    norm = -(jnp.dot(G, dis, preferred_element_type=f32) *
             jnp.dot(S, dis, preferred_element_type=f32))

    def lhat(y):
        t = norm * jnp.dot(G, y, preferred_element_type=f32)
        return lax.dot_general(S, t, (((0,), (0,)), ((), ())),
                               preferred_element_type=f32)

    tx0 = x_ref[...]
    tx1 = lhat(tx0)
    tx2 = 2.0 * lhat(tx1) - tx0
    tx3 = 2.0 * lhat(tx2) - tx1
    tx4 = 2.0 * lhat(tx3) - tx2
    txcat = jnp.concatenate([tx0, tx1, tx2, tx3, tx4], axis=1)
    xc = jnp.dot(txcat, chebw_ref[...],
                 preferred_element_type=f32) + chebb_ref[...]

    ycat_ref[...] = jnp.dot(xc, m3_ref[...], preferred_element_type=f32)
    base_ref[...] = jnp.dot(xc, root_ref[...],
                            preferred_element_type=f32) + bias_ref[...]
    ea5 = ea5_ref[...]
    easpl_ref[...] = jnp.concatenate(
        [jnp.broadcast_to(ea5[:, a:a + 1], (E, L)) for a in range(NA)],
        axis=1)


def _mid_body(base_ref, msg_ref, dst_ref, gamma_ref, beta_ref, m3_ref,
              root_ref, bias_ref, ycat_ref, baseo_ref):
    f32 = jnp.float32
    E = msg_ref.shape[0]
    N = base_ref.shape[0]
    col = lax.broadcasted_iota(jnp.int32, (E, N), 1)
    S = (dst_ref[...] == col).astype(f32)
    agg = lax.dot_general(S, msg_ref[...], (((0,), (0,)), ((), ())),
                          preferred_element_type=f32)
    xl = base_ref[...] + agg
    mu = jnp.mean(xl, axis=1, keepdims=True)
    var = jnp.mean((xl - mu) ** 2, axis=1, keepdims=True)
    h = jnp.maximum((xl - mu) * lax.rsqrt(var + 1e-5) * gamma_ref[...]
                    + beta_ref[...], 0.0)
    ycat_ref[...] = jnp.dot(h, m3_ref[...], preferred_element_type=f32)
    baseo_ref[...] = xl + jnp.dot(h, root_ref[...],
                                  preferred_element_type=f32) + bias_ref[...]


def _final_body(base_ref, msg_ref, dst_ref, gamma_ref, beta_ref, linw_ref,
                linb_ref, out_ref):
    f32 = jnp.float32
    E = msg_ref.shape[0]
    N = base_ref.shape[0]
    col = lax.broadcasted_iota(jnp.int32, (E, N), 1)
    S = (dst_ref[...] == col).astype(f32)
    agg = lax.dot_general(S, msg_ref[...], (((0,), (0,)), ((), ())),
                          preferred_element_type=f32)
    xl = base_ref[...] + agg
    mu = jnp.mean(xl, axis=1, keepdims=True)
    var = jnp.mean((xl - mu) ** 2, axis=1, keepdims=True)
    h = jnp.maximum((xl - mu) * lax.rsqrt(var + 1e-5) * gamma_ref[...]
                    + beta_ref[...], 0.0)
    out_ref[...] = jnp.dot(h, linw_ref[...],
                           preferred_element_type=f32) + linb_ref[...]


def kernel(x, edge_index, edge_attr, batch, params):
    N = x.shape[0]
    E = edge_index.shape[1]
    layers = params['layers']
    f32 = jnp.float32

    upad = jnp.zeros((8, H), f32)
    upad = upad.at[0:4, :].set(params['enc_W'])
    upad = upad.at[4, :].set(params['enc_b'])
    rowmask = jnp.zeros((8, 1), f32).at[4, 0].set(1.0)
    m3 = []
    for l in layers:
        b1pad = jnp.zeros((8, H), f32).at[4, :].set(l['nn_b1'])
        m = _prep_layer(upad, l['nn_W1'], b1pad, rowmask,
                        l['nn_W2'], l['nn_b2'].reshape(1, H * H))
        mt = jnp.transpose(m[:NA].reshape(NA, H, H), (1, 0, 2)).reshape(H, YW)
        m3.append(jnp.concatenate([mt, jnp.zeros((H, YWP - YW), f32)], axis=1))

    src = edge_index[0].astype(jnp.int32)
    dst2d = edge_index[1].astype(jnp.int32).reshape(E, 1)
    ea5 = jnp.concatenate([edge_attr, jnp.ones((E, 1), f32)], axis=1)
    chebw = params['cheb_W'].reshape(NA * 4, H)
    chebb = params['cheb_b'].reshape(1, H)

    l0 = layers[0]
    ycat, base, easpl = pl.pallas_call(
        _cheb_body,
        out_shape=[jax.ShapeDtypeStruct((N, YWP), f32),
                   jax.ShapeDtypeStruct((N, H), f32),
                   jax.ShapeDtypeStruct((E, NA * L), f32)],
    )(x, src.reshape(E, 1), dst2d, ea5, chebw, chebb, m3[0],
      l0['root'], l0['bias'].reshape(1, H))

    for l in (1, 2, 3):
        msg = _edge_sc(ycat, src, easpl, E)
        ll = layers[l]
        ycat, base = pl.pallas_call(
            _mid_body,
            out_shape=[jax.ShapeDtypeStruct((N, YWP), f32),
                       jax.ShapeDtypeStruct((N, H), f32)],
        )(base, msg, dst2d, ll['gamma'].reshape(1, H),
          ll['beta'].reshape(1, H), m3[l], ll['root'],
          ll['bias'].reshape(1, H))

    msg = _edge_sc(ycat, src, easpl, E)
    return pl.pallas_call(
        _final_body,
        out_shape=jax.ShapeDtypeStruct((N, 2), f32),
    )(base, msg, dst2d, l0['gamma'].reshape(1, H), l0['beta'].reshape(1, H),
      params['lin_W'], params['lin_b'].reshape(1, 2))


# speedup vs baseline: 1.7917x; 1.6728x over previous
import functools

import jax
import jax.numpy as jnp
from jax.experimental import pallas as pl

H = 192
NA = 5


def _prep_body(u_ref, w1_ref, b1p_ref, rmask_ref, w2_ref, b2_ref, out_ref):
    a5 = jnp.dot(u_ref[...], w1_ref[...],
                 preferred_element_type=jnp.float32) + b1p_ref[...]
    m = jnp.dot(a5, w2_ref[...], preferred_element_type=jnp.float32)
    out_ref[...] = m + rmask_ref[...] * b2_ref[...]


def _prep_layer(upad, w1, b1pad, rowmask, w2, b2row):
    nb = 8
    bc = (H * H) // nb
    return pl.pallas_call(
        _prep_body,
        grid=(nb,),
        in_specs=[
            pl.BlockSpec((8, H), lambda j: (0, 0)),
            pl.BlockSpec((H, H), lambda j: (0, 0)),
            pl.BlockSpec((8, H), lambda j: (0, 0)),
            pl.BlockSpec((8, 1), lambda j: (0, 0)),
            pl.BlockSpec((H, bc), lambda j: (0, j)),
            pl.BlockSpec((1, bc), lambda j: (0, j)),
        ],
        out_specs=pl.BlockSpec((8, bc), lambda j: (0, j)),
        out_shape=jax.ShapeDtypeStruct((8, H * H), jnp.float32),
    )(upad, w1, b1pad, rowmask, w2, b2row)


def _main_body(x_ref, src_ref, dst_ref, ea5_ref, chebw_ref, chebb_ref,
               m3_ref, roots_ref, biases_ref, gammas_ref, betas_ref,
               linw_ref, linb_ref, out_ref):
    E = src_ref.shape[0]
    N = x_ref.shape[0]
    f32 = jnp.float32

    col = jax.lax.broadcasted_iota(jnp.int32, (E, N), 1)
    G = (src_ref[...] == col).astype(f32)
    S = (dst_ref[...] == col).astype(f32)

    deg = jnp.sum(G, axis=0).reshape(N, 1)
    dis = jnp.where(deg > 0, jax.lax.rsqrt(jnp.maximum(deg, 1e-12)), 0.0)
    dis_src = jnp.dot(G, dis, preferred_element_type=f32)
    dis_dst = jnp.dot(S, dis, preferred_element_type=f32)
    norm = -(dis_src * dis_dst)

    def lhat(y):
        t = norm * jnp.dot(G, y, preferred_element_type=f32)
        return jax.lax.dot_general(S, t, (((0,), (0,)), ((), ())),
                                   preferred_element_type=f32)

    tx0 = x_ref[...]
    tx1 = lhat(tx0)
    tx2 = 2.0 * lhat(tx1) - tx0
    tx3 = 2.0 * lhat(tx2) - tx1
    tx4 = 2.0 * lhat(tx3) - tx2
    txcat = jnp.concatenate([tx0, tx1, tx2, tx3, tx4], axis=1)
    xl = jnp.dot(txcat, chebw_ref[...],
                 preferred_element_type=f32) + chebb_ref[...]

    ea5 = ea5_ref[...]
    bf16 = jnp.bfloat16
    Gb = G.astype(bf16)
    Sb = S.astype(bf16)

    def nnconv(v, l):
        ycat = jnp.dot(v, m3_ref[l], preferred_element_type=f32)
        z = jnp.dot(Gb, ycat.astype(bf16), preferred_element_type=f32)
        msg = ea5[:, 0:1] * z[:, 0:H]
        for a in range(1, NA):
            msg = msg + ea5[:, a:a + 1] * z[:, a * H:(a + 1) * H]
        agg = jax.lax.dot_general(Sb, msg.astype(bf16),
                                  (((0,), (0,)), ((), ())),
                                  preferred_element_type=f32)
        return agg + jnp.dot(v, roots_ref[l],
                             preferred_element_type=f32) + biases_ref[l:l + 1, :]

    def ln_relu(v, l):
        mu = jnp.mean(v, axis=1, keepdims=True)
        var = jnp.mean((v - mu) ** 2, axis=1, keepdims=True)
        nrm = (v - mu) * jax.lax.rsqrt(var + 1e-5)
        return jnp.maximum(nrm * gammas_ref[l:l + 1, :] + betas_ref[l:l + 1, :],
                           0.0)

    xl = nnconv(xl, 0)
    for l in (1, 2, 3):
        xl = xl + nnconv(ln_relu(xl, l), l)
    h = ln_relu(xl, 0)
    out_ref[...] = jnp.dot(h, linw_ref[...],
                           preferred_element_type=f32) + linb_ref[...]


def kernel(x, edge_index, edge_attr, batch, params):
    N = x.shape[0]
    E = edge_index.shape[1]
    layers = params['layers']

    upad = jnp.zeros((8, H), jnp.float32)
    upad = upad.at[0:4, :].set(params['enc_W'])
    upad = upad.at[4, :].set(params['enc_b'])
    rowmask = jnp.zeros((8, 1), jnp.float32).at[4, 0].set(1.0)
    m3cats = []
    for l in layers:
        b1pad = jnp.zeros((8, H), jnp.float32).at[4, :].set(l['nn_b1'])
        m = _prep_layer(upad, l['nn_W1'], b1pad, rowmask,
                        l['nn_W2'], l['nn_b2'].reshape(1, H * H))
        m3cats.append(jnp.transpose(m[:NA].reshape(NA, H, H),
                                    (1, 0, 2)).reshape(H, NA * H))
    m3 = jnp.stack(m3cats)
    roots = jnp.stack([l['root'] for l in layers])
    biases = jnp.stack([l['bias'] for l in layers])
    gammas = jnp.stack([l['gamma'] for l in layers])
    betas = jnp.stack([l['beta'] for l in layers])

    src = edge_index[0].astype(jnp.int32).reshape(E, 1)
    dst = edge_index[1].astype(jnp.int32).reshape(E, 1)
    ea5 = jnp.concatenate([edge_attr, jnp.ones((E, 1), jnp.float32)], axis=1)
    chebw = params['cheb_W'].reshape(NA * 4, H)
    chebb = params['cheb_b'].reshape(1, H)
    linw = params['lin_W']
    linb = params['lin_b'].reshape(1, 2)

    return pl.pallas_call(
        _main_body,
        out_shape=jax.ShapeDtypeStruct((N, 2), jnp.float32),
    )(x, src, dst, ea5, chebw, chebb, m3, roots, biases, gammas, betas,
      linw, linb)


# speedup vs baseline: 1.8268x; 1.0196x over previous
import functools

import jax
import jax.numpy as jnp
from jax.experimental import pallas as pl

H = 192
NA = 5


def _prep_body(u_ref, w1_ref, b1p_ref, rmask_ref, w2_ref, b2_ref, out_ref):
    a5 = jnp.dot(u_ref[...], w1_ref[...],
                 preferred_element_type=jnp.float32) + b1p_ref[...]
    m = jnp.dot(a5.astype(jnp.bfloat16), w2_ref[...].astype(jnp.bfloat16),
                preferred_element_type=jnp.float32)
    out_ref[...] = m + rmask_ref[...] * b2_ref[...]


def _prep_layer(upad, w1, b1pad, rowmask, w2, b2row):
    nb = 4
    bc = (H * H) // nb
    return pl.pallas_call(
        _prep_body,
        grid=(nb,),
        in_specs=[
            pl.BlockSpec((8, H), lambda j: (0, 0)),
            pl.BlockSpec((H, H), lambda j: (0, 0)),
            pl.BlockSpec((8, H), lambda j: (0, 0)),
            pl.BlockSpec((8, 1), lambda j: (0, 0)),
            pl.BlockSpec((H, bc), lambda j: (0, j)),
            pl.BlockSpec((1, bc), lambda j: (0, j)),
        ],
        out_specs=pl.BlockSpec((8, bc), lambda j: (0, j)),
        out_shape=jax.ShapeDtypeStruct((8, H * H), jnp.float32),
    )(upad, w1, b1pad, rowmask, w2, b2row)


def _main_body(x_ref, src_ref, dst_ref, ea5_ref, chebw_ref, chebb_ref,
               m3_ref, roots_ref, biases_ref, gammas_ref, betas_ref,
               linw_ref, linb_ref, out_ref):
    E = src_ref.shape[0]
    N = x_ref.shape[0]
    f32 = jnp.float32

    col = jax.lax.broadcasted_iota(jnp.int32, (E, N), 1)
    G = (src_ref[...] == col).astype(f32)
    S = (dst_ref[...] == col).astype(f32)

    deg = jnp.sum(G, axis=0).reshape(N, 1)
    dis = jnp.where(deg > 0, jax.lax.rsqrt(jnp.maximum(deg, 1e-12)), 0.0)
    dis_src = jnp.dot(G, dis, preferred_element_type=f32)
    dis_dst = jnp.dot(S, dis, preferred_element_type=f32)
    norm = -(dis_src * dis_dst)

    def lhat(y):
        t = norm * jnp.dot(G, y, preferred_element_type=f32)
        return jax.lax.dot_general(S, t, (((0,), (0,)), ((), ())),
                                   preferred_element_type=f32)

    tx0 = x_ref[...]
    tx1 = lhat(tx0)
    tx2 = 2.0 * lhat(tx1) - tx0
    tx3 = 2.0 * lhat(tx2) - tx1
    tx4 = 2.0 * lhat(tx3) - tx2
    txcat = jnp.concatenate([tx0, tx1, tx2, tx3, tx4], axis=1)
    xl = jnp.dot(txcat, chebw_ref[...],
                 preferred_element_type=f32) + chebb_ref[...]

    ea5 = ea5_ref[...]
    bf16 = jnp.bfloat16
    Gb = G.astype(bf16)
    Sb = S.astype(bf16)

    def nnconv(v, l):
        ycat = jnp.dot(v, m3_ref[l], preferred_element_type=f32)
        z = jnp.dot(Gb, ycat.astype(bf16), preferred_element_type=f32)
        msg = ea5[:, 0:1] * z[:, 0:H]
        for a in range(1, NA):
            msg = msg + ea5[:, a:a + 1] * z[:, a * H:(a + 1) * H]
        agg = jax.lax.dot_general(Sb, msg.astype(bf16),
                                  (((0,), (0,)), ((), ())),
                                  preferred_element_type=f32)
        return agg + jnp.dot(v, roots_ref[l],
                             preferred_element_type=f32) + biases_ref[l:l + 1, :]

    def ln_relu(v, l):
        mu = jnp.mean(v, axis=1, keepdims=True)
        var = jnp.mean((v - mu) ** 2, axis=1, keepdims=True)
        nrm = (v - mu) * jax.lax.rsqrt(var + 1e-5)
        return jnp.maximum(nrm * gammas_ref[l:l + 1, :] + betas_ref[l:l + 1, :],
                           0.0)

    xl = nnconv(xl, 0)
    for l in (1, 2, 3):
        xl = xl + nnconv(ln_relu(xl, l), l)
    h = ln_relu(xl, 0)
    out_ref[...] = jnp.dot(h, linw_ref[...],
                           preferred_element_type=f32) + linb_ref[...]


def kernel(x, edge_index, edge_attr, batch, params):
    N = x.shape[0]
    E = edge_index.shape[1]
    layers = params['layers']

    upad = jnp.zeros((8, H), jnp.float32)
    upad = upad.at[0:4, :].set(params['enc_W'])
    upad = upad.at[4, :].set(params['enc_b'])
    rowmask = jnp.zeros((8, 1), jnp.float32).at[4, 0].set(1.0)
    m3cats = []
    for l in layers:
        b1pad = jnp.zeros((8, H), jnp.float32).at[4, :].set(l['nn_b1'])
        m = _prep_layer(upad, l['nn_W1'], b1pad, rowmask,
                        l['nn_W2'], l['nn_b2'].reshape(1, H * H))
        m3cats.append(jnp.transpose(m[:NA].reshape(NA, H, H),
                                    (1, 0, 2)).reshape(H, NA * H))
    m3 = jnp.stack(m3cats)
    roots = jnp.stack([l['root'] for l in layers])
    biases = jnp.stack([l['bias'] for l in layers])
    gammas = jnp.stack([l['gamma'] for l in layers])
    betas = jnp.stack([l['beta'] for l in layers])

    src = edge_index[0].astype(jnp.int32).reshape(E, 1)
    dst = edge_index[1].astype(jnp.int32).reshape(E, 1)
    ea5 = jnp.concatenate([edge_attr, jnp.ones((E, 1), jnp.float32)], axis=1)
    chebw = params['cheb_W'].reshape(NA * 4, H)
    chebb = params['cheb_b'].reshape(1, H)
    linw = params['lin_W']
    linb = params['lin_b'].reshape(1, 2)

    return pl.pallas_call(
        _main_body,
        out_shape=jax.ShapeDtypeStruct((N, 2), jnp.float32),
    )(x, src, dst, ea5, chebw, chebb, m3, roots, biases, gammas, betas,
      linw, linb)
